# baseline (speedup 1.0000x reference)
import functools

import jax
import jax.numpy as jnp
from jax.experimental import pallas as pl
from jax.experimental.pallas import tpu as pltpu

N = 10000
E = 32000
D = 2560
H = 8
C = 320
HC = H * C
G = 8

NPAD = 10240
MBLK = 256


def _mm_body(x_ref, wl_ref, wr_ref, bl_ref, br_ref, xl_ref, xr_ref):
    x = x_ref[...]
    xl_ref[...] = (
        jax.lax.dot(x, wl_ref[...], preferred_element_type=jnp.float32)
        + bl_ref[...]
    )
    xr_ref[...] = (
        jax.lax.dot(x, wr_ref[...], preferred_element_type=jnp.float32)
        + br_ref[...]
    )


def _proj(x_bf16, wl, wr, bl, br):
    k = x_bf16.shape[1]
    grid = (NPAD // MBLK,)
    return pl.pallas_call(
        _mm_body,
        grid=grid,
        in_specs=[
            pl.BlockSpec((MBLK, k), lambda i: (i, 0)),
            pl.BlockSpec((k, HC), lambda i: (0, 0)),
            pl.BlockSpec((k, HC), lambda i: (0, 0)),
            pl.BlockSpec((1, HC), lambda i: (0, 0)),
            pl.BlockSpec((1, HC), lambda i: (0, 0)),
        ],
        out_specs=[
            pl.BlockSpec((MBLK, HC), lambda i: (i, 0)),
            pl.BlockSpec((MBLK, HC), lambda i: (i, 0)),
        ],
        out_shape=[
            jax.ShapeDtypeStruct((NPAD, HC), jnp.float32),
            jax.ShapeDtypeStruct((NPAD, HC), jnp.float32),
        ],
    )(x_bf16, wl, wr, bl[None, :], br[None, :])


def _edge_phase(xl, xr, src, dst, edge_attr, We, att, bias):
    e = edge_attr @ We
    m = xl[src] + xr[dst] + e
    m = jax.nn.leaky_relu(m, negative_slope=0.2)
    m = m.reshape(E, H, C)
    alpha = (m * att[None, :, :]).sum(-1)
    alpha = jnp.exp(alpha)
    asum = jax.ops.segment_sum(alpha, dst, num_segments=N)
    alpha = alpha / (asum[dst] + 1e-16)
    msg = xl[src].reshape(E, H, C) * alpha[:, :, None]
    out = jax.ops.segment_sum(msg, dst, num_segments=N)
    return out.reshape(N, HC) + bias


def kernel(x, edge_index, edge_attr, batch, Wl1, bl1, Wr1, br1, We1, att1, b1,
           Wl2, bl2, Wr2, br2, We2, att2, b2):
    src = edge_index[0]
    dst = edge_index[1]

    xp = jnp.zeros((NPAD, D), jnp.bfloat16).at[:N].set(x.astype(jnp.bfloat16))
    xl1, xr1 = _proj(xp, Wl1.astype(jnp.bfloat16), Wr1.astype(jnp.bfloat16),
                     bl1, br1)
    h = _edge_phase(xl1[:N], xr1[:N], src, dst, edge_attr, We1, att1, b1)

    hp = jnp.zeros((NPAD, HC), jnp.bfloat16).at[:N].set(h.astype(jnp.bfloat16))
    xl2, xr2 = _proj(hp, Wl2.astype(jnp.bfloat16), Wr2.astype(jnp.bfloat16),
                     bl2, br2)
    h2 = _edge_phase(xl2[:N], xr2[:N], src, dst, edge_attr, We2, att2, b2)

    sums = jax.ops.segment_sum(h2, batch, num_segments=G)
    cnt = jax.ops.segment_sum(jnp.ones((N,), jnp.float32), batch,
                              num_segments=G)
    return sums / jnp.clip(cnt, 1.0, None)[:, None]


# speedup vs baseline: 1.4937x; 1.4937x over previous
import functools

import jax
import jax.numpy as jnp
from jax import lax
from jax.experimental import pallas as pl
from jax.experimental.pallas import tpu as pltpu
from jax.experimental.pallas import tpu_sc as plsc

N = 10000
E = 32000
D = 2560
H = 8
C = 320
HC = H * C
G = 8

NPAD = 10240
MBLK = 256

B = 8
NB = N // B
NBP = NPAD // B
GSZ = 16
EP = E + GSZ * NB
NCHUNK = HC // 16
CPH = C // 16


def _mm_body(x_ref, wl_ref, wr_ref, bl_ref, br_ref, xl_ref, xr_ref):
    x = x_ref[...]
    xl_ref[...] = (
        jax.lax.dot(x, wl_ref[...], preferred_element_type=jnp.float32)
        + bl_ref[...]
    )
    xr_ref[...] = (
        jax.lax.dot(x, wr_ref[...], preferred_element_type=jnp.float32)
        + br_ref[...]
    )


def _proj(x_bf16, wl, wr, bl, br):
    k = x_bf16.shape[1]
    return pl.pallas_call(
        _mm_body,
        grid=(NPAD // MBLK,),
        in_specs=[
            pl.BlockSpec((MBLK, k), lambda i: (i, 0)),
            pl.BlockSpec((k, HC), lambda i: (0, 0)),
            pl.BlockSpec((k, HC), lambda i: (0, 0)),
            pl.BlockSpec((1, HC), lambda i: (0, 0)),
            pl.BlockSpec((1, HC), lambda i: (0, 0)),
        ],
        out_specs=[
            pl.BlockSpec((MBLK, HC), lambda i: (i, 0)),
            pl.BlockSpec((MBLK, HC), lambda i: (i, 0)),
        ],
        out_shape=[
            jax.ShapeDtypeStruct((NPAD, HC), jnp.float32),
            jax.ShapeDtypeStruct((NPAD, HC), jnp.float32),
        ],
    )(x_bf16, wl, wr, bl[None, :], br[None, :])


def _sc_edge_body(xl_h, xr_h, psrc_h, pdl_h, pea_h, est2_h, wab_h, out_h,
                  est_s, xr_s, acc_s, asum_s, rec_s, xlg_s, idx_s, dl_s,
                  ea_s, wab_s, gsem):
    ncores = 2
    nw = 32
    nbt = NBP // nw

    wid = lax.axis_index("s") * ncores + lax.axis_index("c")

    pltpu.sync_copy(est2_h, est_s)
    pltpu.sync_copy(wab_h, wab_s)

    zero16 = jnp.zeros((16,), jnp.float32)

    def _zxr(k, _):
        xr_s[B, pl.ds(k * 16, 16)] = zero16
        return 0
    lax.fori_loop(0, NCHUNK, _zxr, 0)

    lane = lax.broadcasted_iota(jnp.int32, (GSZ,), 0)

    def block_body(bi, _):
        b = bi * nw + wid
        ev = est_s[b]
        p0 = ev[0]
        ngb = ev[1]
        realf = ev[2].astype(jnp.float32)
        n0 = b * B

        def _zacc(r, _):
            def _zc(k, __):
                acc_s[r, pl.ds(k * 16, 16)] = zero16
                return 0
            lax.fori_loop(0, NCHUNK, _zc, 0)
            asum_s[r] = zero16
            return 0
        lax.fori_loop(0, B + 1, _zacc, 0)

        pltpu.sync_copy(xr_h.at[pl.ds(n0, B)], xr_s.at[pl.ds(0, B)])

        def group_body(g, _):
            base = pl.multiple_of(p0 + g * GSZ, GSZ)
            pltpu.sync_copy(psrc_h.at[pl.ds(base, GSZ)], idx_s)
            pltpu.sync_copy(pdl_h.at[pl.ds(base, GSZ)], dl_s)
            pltpu.sync_copy(pea_h.at[pl.ds(base, GSZ)], ea_s)
            pltpu.async_copy(xl_h.at[idx_s], xlg_s, gsem).wait()

            def edge_body(i, _):
                iv = jnp.full((GSZ,), i, jnp.int32)
                dl = plsc.load_gather(dl_s, [iv])[0]
                ea = plsc.load_gather(ea_s, [iv])[0]

                def head_body(h, _):
                    hoff = h * C

                    def _logit(k, accv):
                        o = hoff + k * 16
                        z = (xlg_s[i, pl.ds(o, 16)] + xr_s[dl, pl.ds(o, 16)]
                             + ea * wab_s[0, pl.ds(o, 16)])
                        p = jnp.maximum(z, 0.2 * z)
                        return accv + p * wab_s[1, pl.ds(o, 16)]
                    accv = lax.fori_loop(0, CPH, _logit, zero16)
                    lh = jnp.sum(accv)
                    ah = jnp.exp(jnp.full((GSZ,), lh, jnp.float32))[0]

                    def _msg(k, __):
                        o = hoff + k * 16
                        acc_s[dl, pl.ds(o, 16)] += ah * xlg_s[i, pl.ds(o, 16)]
                        return 0
                    lax.fori_loop(0, CPH, _msg, 0)
                    asum_s[dl] += jnp.where(lane == h, ah, 0.0)
                    return 0
                lax.fori_loop(0, H, head_body, 0)
                return 0
            lax.fori_loop(0, GSZ, edge_body, 0)
            return 0
        lax.fori_loop(0, ngb, group_body, 0)

        def _rec(r, _):
            rec_s[r] = 1.0 / (asum_s[r] + 1e-16)
            return 0
        lax.fori_loop(0, B, _rec, 0)

        def _norm_r(r, _):
            rv16 = jnp.full((GSZ,), r, jnp.int32)

            def _norm_h(h, __):
                rv = plsc.load_gather(rec_s, [rv16, jnp.full((GSZ,), h,
                                                             jnp.int32)])[0]
                hoff = h * C

                def _norm_k(k, ___):
                    o = hoff + k * 16
                    acc_s[r, pl.ds(o, 16)] = (acc_s[r, pl.ds(o, 16)] * rv
                                              + realf * wab_s[2, pl.ds(o, 16)])
                    return 0
                lax.fori_loop(0, CPH, _norm_k, 0)
                return 0
            lax.fori_loop(0, H, _norm_h, 0)
            return 0
        lax.fori_loop(0, B, _norm_r, 0)

        pltpu.sync_copy(acc_s.at[pl.ds(0, B)], out_h.at[pl.ds(n0, B)])
        return 0
    lax.fori_loop(0, nbt, block_body, 0)


def _sc_edge(xl, xr, psrc, pdl, pea, est2, wab):
    mesh = plsc.VectorSubcoreMesh(core_axis_name="c", subcore_axis_name="s")
    return pl.kernel(
        _sc_edge_body,
        out_type=jax.ShapeDtypeStruct((NPAD, HC), jnp.float32),
        mesh=mesh,
        compiler_params=pltpu.CompilerParams(needs_layout_passes=False,
                                             use_tc_tiling_on_sc=False),
        scratch_types=[
            pltpu.VMEM((NBP, 16), jnp.int32),
            pltpu.VMEM((B + 1, HC), jnp.float32),
            pltpu.VMEM((B + 1, HC), jnp.float32),
            pltpu.VMEM((B + 1, 16), jnp.float32),
            pltpu.VMEM((B, 16), jnp.float32),
            pltpu.VMEM((GSZ, HC), jnp.float32),
            pltpu.VMEM((GSZ,), jnp.int32),
            pltpu.VMEM((GSZ,), jnp.int32),
            pltpu.VMEM((GSZ,), jnp.float32),
            pltpu.VMEM((3, HC), jnp.float32),
            pltpu.SemaphoreType.DMA,
        ],
    )(xl, xr, psrc, pdl, pea, est2, wab)


def _pool_body(batch_ref, h_ref, sums_ref, cnt_ref, out_ref):
    i = pl.program_id(0)

    @pl.when(i == 0)
    def _init():
        sums_ref[...] = jnp.zeros_like(sums_ref)
        cnt_ref[...] = jnp.zeros_like(cnt_ref)

    bvec = batch_ref[0, 0, :]
    onehot = (bvec[:, None]
              == lax.broadcasted_iota(jnp.int32, (1, G), 1)).astype(
                  jnp.float32)
    sums_ref[...] += lax.dot_general(
        onehot, h_ref[...], (((0,), (0,)), ((), ())),
        preferred_element_type=jnp.float32)
    cnt_ref[...] += jnp.broadcast_to(jnp.sum(onehot, axis=0)[:, None],
                                     (G, 128))

    @pl.when(i == (NPAD // MBLK) - 1)
    def _fin():
        cnt = jnp.clip(cnt_ref[:, 0:1], 1.0, None)
        out_ref[...] = sums_ref[...] / cnt


def _pool(batch3, h2):
    _, _, out = pl.pallas_call(
        _pool_body,
        grid=(NPAD // MBLK,),
        in_specs=[
            pl.BlockSpec((1, 1, MBLK), lambda i: (i, 0, 0)),
            pl.BlockSpec((MBLK, HC), lambda i: (i, 0)),
        ],
        out_specs=[
            pl.BlockSpec((G, HC), lambda i: (0, 0)),
            pl.BlockSpec((G, 128), lambda i: (0, 0)),
            pl.BlockSpec((G, HC), lambda i: (0, 0)),
        ],
        out_shape=[
            jax.ShapeDtypeStruct((G, HC), jnp.float32),
            jax.ShapeDtypeStruct((G, 128), jnp.float32),
            jax.ShapeDtypeStruct((G, HC), jnp.float32),
        ],
    )(batch3, h2)
    return out


def kernel(x, edge_index, edge_attr, batch, Wl1, bl1, Wr1, br1, We1, att1, b1,
           Wl2, bl2, Wr2, br2, We2, att2, b2):
    src = edge_index[0]
    dst = edge_index[1]

    perm = jnp.argsort(dst)
    srcs = src[perm]
    dsts = dst[perm]
    eas = edge_attr[perm, 0]
    blk = dsts // B
    estart = jnp.searchsorted(dsts, (jnp.arange(NB + 1) * B).astype(jnp.int32)
                              ).astype(jnp.int32)
    cnt = estart[1:] - estart[:-1]
    ngrp = (cnt + GSZ - 1) // GSZ
    pcnt = ngrp * GSZ
    pstart = jnp.concatenate([jnp.zeros((1,), jnp.int32),
                              jnp.cumsum(pcnt).astype(jnp.int32)])
    pos = pstart[blk] + (jnp.arange(E, dtype=jnp.int32) - estart[blk])
    psrc = jnp.zeros((EP,), jnp.int32).at[pos].set(srcs)
    pdl = jnp.full((EP,), B, jnp.int32).at[pos].set(
        (dsts - blk * B).astype(jnp.int32))
    pea = jnp.zeros((EP,), jnp.float32).at[pos].set(eas)
    est2 = jnp.zeros((NBP, 16), jnp.int32)
    est2 = (est2.at[:NB, 0].set(pstart[:NB])
                .at[:NB, 1].set(ngrp)
                .at[:NB, 2].set(1))

    wab1 = jnp.stack([We1[0], att1.reshape(-1), b1])
    wab2 = jnp.stack([We2[0], att2.reshape(-1), b2])

    xp = jnp.zeros((NPAD, D), jnp.bfloat16).at[:N].set(x.astype(jnp.bfloat16))
    xl1, xr1 = _proj(xp, Wl1.astype(jnp.bfloat16), Wr1.astype(jnp.bfloat16),
                     bl1, br1)
    h = _sc_edge(xl1, xr1, psrc, pdl, pea, est2, wab1)

    xl2, xr2 = _proj(h.astype(jnp.bfloat16), Wl2.astype(jnp.bfloat16),
                     Wr2.astype(jnp.bfloat16), bl2, br2)
    h2 = _sc_edge(xl2, xr2, psrc, pdl, pea, est2, wab2)

    batch3 = jnp.full((NPAD,), G, jnp.int32).at[:N].set(batch).reshape(
        NPAD // MBLK, 1, MBLK)
    return _pool(batch3, h2)


# speedup vs baseline: 2.0800x; 1.3925x over previous
import numpy as np

import jax
import jax.numpy as jnp
from jax import lax
from jax.experimental import pallas as pl
from jax.experimental.pallas import tpu as pltpu
from jax.experimental.pallas import tpu_sc as plsc

N = 10000
E = 32000
D = 2560
H = 8
C = 320
HC = H * C
G = 8

NPAD = 10240
MBLK = 256

B = 8
NB = N // B
NBP = NPAD // B
GSZ = 16
EP = E + GSZ * NB
EPG = EP // GSZ
NCHUNK = HC // 16
CP2 = C // 32

_p = np.arange(HC)
IPERM = ((_p // 32) * 32 + (_p % 2) * 16 + (_p % 32) // 2).astype(np.int32)
INV_IPERM = np.argsort(IPERM).astype(np.int32)

_ILV = plsc.PackFormat.INTERLEAVED


def _mm_body(x_ref, wl_ref, wr_ref, bl_ref, br_ref, xl_ref, xr_ref):
    x = x_ref[...]
    xl_ref[...] = (
        jax.lax.dot(x, wl_ref[...], preferred_element_type=jnp.float32)
        + bl_ref[...]
    ).astype(jnp.bfloat16)
    xr_ref[...] = (
        jax.lax.dot(x, wr_ref[...], preferred_element_type=jnp.float32)
        + br_ref[...]
    ).astype(jnp.bfloat16)


def _proj(x_bf16, wl, wr, bl, br):
    k = x_bf16.shape[1]
    return pl.pallas_call(
        _mm_body,
        grid=(NPAD // MBLK,),
        in_specs=[
            pl.BlockSpec((MBLK, k), lambda i: (i, 0)),
            pl.BlockSpec((k, HC), lambda i: (0, 0)),
            pl.BlockSpec((k, HC), lambda i: (0, 0)),
            pl.BlockSpec((1, HC), lambda i: (0, 0)),
            pl.BlockSpec((1, HC), lambda i: (0, 0)),
        ],
        out_specs=[
            pl.BlockSpec((MBLK, HC), lambda i: (i, 0)),
            pl.BlockSpec((MBLK, HC), lambda i: (i, 0)),
        ],
        out_shape=[
            jax.ShapeDtypeStruct((NPAD, HC), jnp.bfloat16),
            jax.ShapeDtypeStruct((NPAD, HC), jnp.bfloat16),
        ],
    )(x_bf16, wl, wr, bl[None, :], br[None, :])


def _sc_edge_body(xl_h, xr_h, pmeta_h, est2_h, wat_h, bias_h, out_h,
                  est_s, xr_s, acc_s, asum_s, rec_s, xlg_s, meta_s, stage_s,
                  wat_s, bias_s, gsem):
    ncores = 2
    nw = 32
    nbt = NBP // nw

    wid = lax.axis_index("s") * ncores + lax.axis_index("c")

    pltpu.sync_copy(est2_h, est_s)
    pltpu.sync_copy(wat_h, wat_s)
    pltpu.sync_copy(bias_h, bias_s)

    zero16 = jnp.zeros((16,), jnp.float32)
    zero32b = jnp.zeros((32,), jnp.bfloat16)

    def _zxr(j, _):
        xr_s[B, pl.ds(j * 32, 32)] = zero32b
        return 0
    lax.fori_loop(0, HC // 32, _zxr, 0)

    lane = lax.broadcasted_iota(jnp.int32, (GSZ,), 0)

    def block_body(bi, _):
        b = bi * nw + wid
        ev = est_s[b]
        p0 = ev[0]
        ngb = ev[1]
        realf = ev[2].astype(jnp.float32)
        gi0 = lax.shift_right_logical(p0, 4)
        n0 = b * B

        def _zacc(r, _):
            def _zc(k, __):
                acc_s[r, pl.ds(k * 16, 16)] = zero16
                return 0
            lax.fori_loop(0, NCHUNK, _zc, 0)
            asum_s[r] = zero16
            return 0
        lax.fori_loop(0, B + 1, _zacc, 0)

        pltpu.sync_copy(xr_h.at[pl.ds(n0, B)], xr_s.at[pl.ds(0, B)])

        @pl.when(ngb > 0)
        def _prol():
            pltpu.sync_copy(pmeta_h.at[gi0], meta_s.at[0])
            pltpu.async_copy(xl_h.at[meta_s.at[0, 0]], xlg_s.at[0], gsem)

        def group_body(g, _):
            par = lax.bitwise_and(g, 1)
            pltpu.make_async_copy(xl_h.at[meta_s.at[par, 0]],
                                  xlg_s.at[par], gsem).wait()

            @pl.when(g + 1 < ngb)
            def _prefetch():
                npar = lax.bitwise_and(g + 1, 1)
                pltpu.sync_copy(pmeta_h.at[gi0 + g + 1], meta_s.at[npar])
                pltpu.async_copy(xl_h.at[meta_s.at[npar, 0]],
                                 xlg_s.at[npar], gsem)

            def edge_body(i, _):
                iv = jnp.full((GSZ,), i, jnp.int32)
                dl = plsc.load_gather(meta_s.at[par, 1], [iv])[0]
                eab = plsc.bitcast(plsc.load_gather(meta_s.at[par, 2], [iv]),
                                   jnp.bfloat16)

                def head_body(h, _):
                    hoff = h * C

                    def _logit(j, accv):
                        o = hoff + j * 32
                        z = (xlg_s[par, i, pl.ds(o, 32)]
                             + xr_s[dl, pl.ds(o, 32)]
                             + eab * wat_s[0, pl.ds(o, 32)])
                        z = jnp.maximum(z, 0.2 * z)
                        za, zb = plsc.unpack(z, format=_ILV)
                        aa, ab = plsc.unpack(wat_s[1, pl.ds(o, 32)],
                                             format=_ILV)
                        return accv + za * aa + zb * ab
                    accv = lax.fori_loop(0, CP2, _logit, zero16)
                    lh = jnp.sum(accv)
                    ah = jnp.exp(jnp.full((GSZ,), lh, jnp.float32))[0]

                    def _msg(j, __):
                        o = hoff + j * 32
                        la, lb = plsc.unpack(xlg_s[par, i, pl.ds(o, 32)],
                                             format=_ILV)
                        plsc.addupdate(acc_s.at[dl, pl.ds(o, 16)], ah * la)
                        plsc.addupdate(acc_s.at[dl, pl.ds(o + 16, 16)],
                                       ah * lb)
                        return 0
                    lax.fori_loop(0, CP2, _msg, 0)
                    plsc.addupdate(asum_s.at[dl],
                                   jnp.where(lane == h, ah, 0.0))
                    return 0
                lax.fori_loop(0, H, head_body, 0)
                return 0
            lax.fori_loop(0, GSZ, edge_body, 0)
            return 0
        lax.fori_loop(0, ngb, group_body, 0)

        def _rec(r, _):
            rec_s[r] = 1.0 / (asum_s[r] + 1e-16)
            return 0
        lax.fori_loop(0, B, _rec, 0)

        def _norm_r(r, _):
            rv16 = jnp.full((GSZ,), r, jnp.int32)

            def _norm_h(h, __):
                rv = plsc.load_gather(rec_s, [rv16, jnp.full((GSZ,), h,
                                                             jnp.int32)])[0]
                hoff = h * C

                def _norm_j(j, ___):
                    o = hoff + j * 32
                    va = (acc_s[r, pl.ds(o, 16)] * rv
                          + realf * bias_s[pl.ds(o, 16)])
                    vb = (acc_s[r, pl.ds(o + 16, 16)] * rv
                          + realf * bias_s[pl.ds(o + 16, 16)])
                    stage_s[r, pl.ds(o, 32)] = plsc.pack(va, vb, format=_ILV)
                    return 0
                lax.fori_loop(0, CP2, _norm_j, 0)
                return 0
            lax.fori_loop(0, H, _norm_h, 0)
            return 0
        lax.fori_loop(0, B, _norm_r, 0)

        pltpu.sync_copy(stage_s, out_h.at[pl.ds(n0, B)])
        return 0
    lax.fori_loop(0, nbt, block_body, 0)


def _sc_edge(xl, xr, pmeta, est2, wat, bias):
    mesh = plsc.VectorSubcoreMesh(core_axis_name="c", subcore_axis_name="s")
    return pl.kernel(
        _sc_edge_body,
        out_type=jax.ShapeDtypeStruct((NPAD, HC), jnp.bfloat16),
        mesh=mesh,
        compiler_params=pltpu.CompilerParams(needs_layout_passes=False,
                                             use_tc_tiling_on_sc=False),
        scratch_types=[
            pltpu.VMEM((NBP, 16), jnp.int32),
            pltpu.VMEM((B + 1, HC), jnp.bfloat16),
            pltpu.VMEM((B + 1, HC), jnp.float32),
            pltpu.VMEM((B + 1, 16), jnp.float32),
            pltpu.VMEM((B, 16), jnp.float32),
            pltpu.VMEM((2, GSZ, HC), jnp.bfloat16),
            pltpu.VMEM((2, 3, 16), jnp.int32),
            pltpu.VMEM((B, HC), jnp.bfloat16),
            pltpu.VMEM((2, HC), jnp.bfloat16),
            pltpu.VMEM((HC,), jnp.float32),
            pltpu.SemaphoreType.DMA,
        ],
    )(xl, xr, pmeta, est2, wat, bias)


def _pool_body(batch_ref, h_ref, sums_ref, cnt_ref, out_ref):
    i = pl.program_id(0)

    @pl.when(i == 0)
    def _init():
        sums_ref[...] = jnp.zeros_like(sums_ref)
        cnt_ref[...] = jnp.zeros_like(cnt_ref)

    bvec = batch_ref[0, 0, :]
    onehot = (bvec[:, None]
              == lax.broadcasted_iota(jnp.int32, (1, G), 1)).astype(
                  jnp.bfloat16)
    sums_ref[...] += lax.dot_general(
        onehot, h_ref[...], (((0,), (0,)), ((), ())),
        preferred_element_type=jnp.float32)
    cnt_ref[...] += jnp.broadcast_to(
        jnp.sum(onehot.astype(jnp.float32), axis=0)[:, None], (G, 128))

    @pl.when(i == (NPAD // MBLK) - 1)
    def _fin():
        cnt = jnp.clip(cnt_ref[:, 0:1], 1.0, None)
        out_ref[...] = sums_ref[...] / cnt


def _pool(batch3, h2):
    _, _, out = pl.pallas_call(
        _pool_body,
        grid=(NPAD // MBLK,),
        in_specs=[
            pl.BlockSpec((1, 1, MBLK), lambda i: (i, 0, 0)),
            pl.BlockSpec((MBLK, HC), lambda i: (i, 0)),
        ],
        out_specs=[
            pl.BlockSpec((G, HC), lambda i: (0, 0)),
            pl.BlockSpec((G, 128), lambda i: (0, 0)),
            pl.BlockSpec((G, HC), lambda i: (0, 0)),
        ],
        out_shape=[
            jax.ShapeDtypeStruct((G, HC), jnp.float32),
            jax.ShapeDtypeStruct((G, 128), jnp.float32),
            jax.ShapeDtypeStruct((G, HC), jnp.float32),
        ],
    )(batch3, h2)
    return out


def kernel(x, edge_index, edge_attr, batch, Wl1, bl1, Wr1, br1, We1, att1, b1,
           Wl2, bl2, Wr2, br2, We2, att2, b2):
    src = edge_index[0]
    dst = edge_index[1]

    perm = jnp.argsort(dst)
    srcs = src[perm]
    dsts = dst[perm]
    eas = edge_attr[perm, 0]
    blk = dsts // B
    estart = jnp.searchsorted(dsts, (jnp.arange(NB + 1) * B).astype(jnp.int32)
                              ).astype(jnp.int32)
    cnt = estart[1:] - estart[:-1]
    ngrp = (cnt + GSZ - 1) // GSZ
    pstart = jnp.concatenate([jnp.zeros((1,), jnp.int32),
                              jnp.cumsum(ngrp * GSZ).astype(jnp.int32)])
    pos = pstart[blk] + (jnp.arange(E, dtype=jnp.int32) - estart[blk])
    psrc = jnp.zeros((EP,), jnp.int32).at[pos].set(srcs)
    pdl = jnp.full((EP,), B, jnp.int32).at[pos].set(
        (dsts - blk * B).astype(jnp.int32))
    pea = jnp.zeros((EP,), jnp.float32).at[pos].set(eas)
    eau = lax.bitcast_convert_type(pea.astype(jnp.bfloat16),
                                   jnp.uint16).astype(jnp.uint32)
    pea_bits = (eau | (eau << 16)).astype(jnp.int32)
    pmeta = jnp.concatenate([
        psrc.reshape(EPG, 1, GSZ),
        pdl.reshape(EPG, 1, GSZ),
        pea_bits.reshape(EPG, 1, GSZ),
    ], axis=1)
    est2 = jnp.zeros((NBP, 16), jnp.int32)
    est2 = (est2.at[:NB, 0].set(pstart[:NB])
                .at[:NB, 1].set(ngrp)
                .at[:NB, 2].set(1))

    wat1 = jnp.stack([We1[0, IPERM], att1.reshape(-1)[IPERM]]).astype(
        jnp.bfloat16)
    wat2 = jnp.stack([We2[0, IPERM], att2.reshape(-1)[IPERM]]).astype(
        jnp.bfloat16)

    xp = jnp.zeros((NPAD, D), jnp.bfloat16).at[:N].set(x.astype(jnp.bfloat16))
    xl1, xr1 = _proj(xp,
                     Wl1[:, IPERM].astype(jnp.bfloat16),
                     Wr1[:, IPERM].astype(jnp.bfloat16),
                     bl1[IPERM], br1[IPERM])
    h = _sc_edge(xl1, xr1, pmeta, est2, wat1, b1)

    xl2, xr2 = _proj(h,
                     Wl2[IPERM][:, IPERM].astype(jnp.bfloat16),
                     Wr2[IPERM][:, IPERM].astype(jnp.bfloat16),
                     bl2[IPERM], br2[IPERM])
    h2 = _sc_edge(xl2, xr2, pmeta, est2, wat2, b2)

    batch3 = jnp.full((NPAD,), G, jnp.int32).at[:N].set(batch).reshape(
        NPAD // MBLK, 1, MBLK)
    pooled = _pool(batch3, h2)
    return pooled[:, INV_IPERM]


# speedup vs baseline: 2.2434x; 1.0786x over previous
import numpy as np

import jax
import jax.numpy as jnp
from jax import lax
from jax.experimental import pallas as pl
from jax.experimental.pallas import tpu as pltpu
from jax.experimental.pallas import tpu_sc as plsc

N = 10000
E = 32000
D = 2560
H = 8
C = 320
HC = H * C
G = 8

NPAD = 10240
MBLK = 256

B = 8
NB = N // B
NBP = NPAD // B
GSZ = 16
EP = E + GSZ * NB
EPG = EP // GSZ
NCHUNK = HC // 16
CP2 = C // 32

_p = np.arange(HC)
IPERM = ((_p // 32) * 32 + (_p % 2) * 16 + (_p % 32) // 2).astype(np.int32)
INV_IPERM = np.argsort(IPERM).astype(np.int32)

_ILV = plsc.PackFormat.INTERLEAVED


def _mm_body(x_ref, wl_ref, wr_ref, bl_ref, br_ref, xl_ref, xr_ref):
    x = x_ref[...]
    xl_ref[...] = (
        jax.lax.dot(x, wl_ref[...], preferred_element_type=jnp.float32)
        + bl_ref[...]
    ).astype(jnp.bfloat16)
    xr_ref[...] = (
        jax.lax.dot(x, wr_ref[...], preferred_element_type=jnp.float32)
        + br_ref[...]
    ).astype(jnp.bfloat16)


def _proj(x_bf16, wl, wr, bl, br):
    k = x_bf16.shape[1]
    return pl.pallas_call(
        _mm_body,
        grid=(NPAD // MBLK,),
        in_specs=[
            pl.BlockSpec((MBLK, k), lambda i: (i, 0)),
            pl.BlockSpec((k, HC), lambda i: (0, 0)),
            pl.BlockSpec((k, HC), lambda i: (0, 0)),
            pl.BlockSpec((1, HC), lambda i: (0, 0)),
            pl.BlockSpec((1, HC), lambda i: (0, 0)),
        ],
        out_specs=[
            pl.BlockSpec((MBLK, HC), lambda i: (i, 0)),
            pl.BlockSpec((MBLK, HC), lambda i: (i, 0)),
        ],
        out_shape=[
            jax.ShapeDtypeStruct((NPAD, HC), jnp.bfloat16),
            jax.ShapeDtypeStruct((NPAD, HC), jnp.bfloat16),
        ],
    )(x_bf16, wl, wr, bl[None, :], br[None, :])


def _sc_edge_body(xl_h, xr_h, pmeta_h, est2_h, wat_h, bias_h, out_h,
                  est_s, xr_s, acc_s, asum_s, rec_s, xlg_s, meta_s, stage_s,
                  wat_s, bias_s, gsem):
    ncores = 2
    nw = 32
    nbt = NBP // nw

    wid = lax.axis_index("s") * ncores + lax.axis_index("c")

    pltpu.sync_copy(est2_h, est_s)
    pltpu.sync_copy(wat_h, wat_s)
    pltpu.sync_copy(bias_h, bias_s)

    zero16 = jnp.zeros((16,), jnp.float32)
    zero32b = jnp.zeros((32,), jnp.bfloat16)

    def _zxr(j, _):
        xr_s[B, pl.ds(j * 32, 32)] = zero32b
        return 0
    lax.fori_loop(0, HC // 32, _zxr, 0)

    lane = lax.broadcasted_iota(jnp.int32, (GSZ,), 0)

    def block_body(bi, _):
        b = bi * nw + wid
        ev = est_s[b]
        p0 = ev[0]
        ngb = ev[1]
        realf = ev[2].astype(jnp.float32)
        gi0 = lax.shift_right_logical(p0, 4)
        n0 = b * B

        def _zacc(r, _):
            def _zc(k, __):
                acc_s[r, pl.ds(k * 16, 16)] = zero16
                return 0
            lax.fori_loop(0, NCHUNK, _zc, 0)
            asum_s[r] = zero16
            return 0
        lax.fori_loop(0, B + 1, _zacc, 0)

        pltpu.sync_copy(xr_h.at[pl.ds(n0, B)], xr_s.at[pl.ds(0, B)])

        @pl.when(ngb > 0)
        def _prol():
            pltpu.sync_copy(pmeta_h.at[gi0], meta_s.at[0])
            pltpu.async_copy(xl_h.at[meta_s.at[0, 0]], xlg_s.at[0], gsem)

        def group_body(g, _):
            par = lax.bitwise_and(g, 1)
            pltpu.make_async_copy(xl_h.at[meta_s.at[par, 0]],
                                  xlg_s.at[par], gsem).wait()

            @pl.when(g + 1 < ngb)
            def _prefetch():
                npar = lax.bitwise_and(g + 1, 1)
                pltpu.sync_copy(pmeta_h.at[gi0 + g + 1], meta_s.at[npar])
                pltpu.async_copy(xl_h.at[meta_s.at[npar, 0]],
                                 xlg_s.at[npar], gsem)

            def edge_body(i, _):
                iv = jnp.full((GSZ,), i, jnp.int32)
                dl = plsc.load_gather(meta_s.at[par, 1], [iv])[0]
                eab = plsc.bitcast(plsc.load_gather(meta_s.at[par, 2], [iv]),
                                   jnp.bfloat16)

                def _logit_term(o, accv):
                    z = (xlg_s[par, i, pl.ds(o, 32)]
                         + xr_s[dl, pl.ds(o, 32)]
                         + eab * wat_s[0, pl.ds(o, 32)])
                    z = jnp.maximum(z, 0.2 * z)
                    za, zb = plsc.unpack(z, format=_ILV)
                    aa, ab = plsc.unpack(wat_s[1, pl.ds(o, 32)], format=_ILV)
                    return accv + za * aa + zb * ab

                def _msg_term(o, ah):
                    la, lb = plsc.unpack(xlg_s[par, i, pl.ds(o, 32)],
                                         format=_ILV)
                    plsc.addupdate(acc_s.at[dl, pl.ds(o, 16)], ah * la)
                    plsc.addupdate(acc_s.at[dl, pl.ds(o + 16, 16)], ah * lb)

                def _exp16(accv):
                    return jnp.exp(jnp.full((GSZ,), jnp.sum(accv),
                                            jnp.float32))[0]

                acc0 = lax.fori_loop(
                    0, CP2, lambda j, a: _logit_term(j * 32, a), zero16)
                ah0 = _exp16(acc0)

                def chain(h, ahp):
                    hoff = h * C

                    def _f(j, accv):
                        _msg_term(hoff - C + j * 32, ahp)
                        return _logit_term(hoff + j * 32, accv)
                    accv = lax.fori_loop(0, CP2, _f, zero16)
                    plsc.addupdate(asum_s.at[dl],
                                   jnp.where(lane == h - 1, ahp, 0.0))
                    return _exp16(accv)
                ah7 = lax.fori_loop(1, H, chain, ah0)

                def _m7(j, __):
                    _msg_term((H - 1) * C + j * 32, ah7)
                    return 0
                lax.fori_loop(0, CP2, _m7, 0)
                plsc.addupdate(asum_s.at[dl],
                               jnp.where(lane == H - 1, ah7, 0.0))
                return 0
            lax.fori_loop(0, GSZ, edge_body, 0)
            return 0
        lax.fori_loop(0, ngb, group_body, 0)

        def _rec(r, _):
            rec_s[r] = 1.0 / (asum_s[r] + 1e-16)
            return 0
        lax.fori_loop(0, B, _rec, 0)

        def _norm_r(r, _):
            rv16 = jnp.full((GSZ,), r, jnp.int32)

            def _norm_h(h, __):
                rv = plsc.load_gather(rec_s, [rv16, jnp.full((GSZ,), h,
                                                             jnp.int32)])[0]
                hoff = h * C

                def _norm_j(j, ___):
                    o = hoff + j * 32
                    va = (acc_s[r, pl.ds(o, 16)] * rv
                          + realf * bias_s[pl.ds(o, 16)])
                    vb = (acc_s[r, pl.ds(o + 16, 16)] * rv
                          + realf * bias_s[pl.ds(o + 16, 16)])
                    stage_s[r, pl.ds(o, 32)] = plsc.pack(va, vb, format=_ILV)
                    return 0
                lax.fori_loop(0, CP2, _norm_j, 0)
                return 0
            lax.fori_loop(0, H, _norm_h, 0)
            return 0
        lax.fori_loop(0, B, _norm_r, 0)

        pltpu.sync_copy(stage_s, out_h.at[pl.ds(n0, B)])
        return 0
    lax.fori_loop(0, nbt, block_body, 0)


def _sc_edge(xl, xr, pmeta, est2, wat, bias):
    mesh = plsc.VectorSubcoreMesh(core_axis_name="c", subcore_axis_name="s")
    return pl.kernel(
        _sc_edge_body,
        out_type=jax.ShapeDtypeStruct((NPAD, HC), jnp.bfloat16),
        mesh=mesh,
        compiler_params=pltpu.CompilerParams(needs_layout_passes=False,
                                             use_tc_tiling_on_sc=False),
        scratch_types=[
            pltpu.VMEM((NBP, 16), jnp.int32),
            pltpu.VMEM((B + 1, HC), jnp.bfloat16),
            pltpu.VMEM((B + 1, HC), jnp.float32),
            pltpu.VMEM((B + 1, 16), jnp.float32),
            pltpu.VMEM((B, 16), jnp.float32),
            pltpu.VMEM((2, GSZ, HC), jnp.bfloat16),
            pltpu.VMEM((2, 3, 16), jnp.int32),
            pltpu.VMEM((B, HC), jnp.bfloat16),
            pltpu.VMEM((2, HC), jnp.bfloat16),
            pltpu.VMEM((HC,), jnp.float32),
            pltpu.SemaphoreType.DMA,
        ],
    )(xl, xr, pmeta, est2, wat, bias)


def _pool_body(batch_ref, h_ref, sums_ref, cnt_ref, out_ref):
    i = pl.program_id(0)

    @pl.when(i == 0)
    def _init():
        sums_ref[...] = jnp.zeros_like(sums_ref)
        cnt_ref[...] = jnp.zeros_like(cnt_ref)

    bvec = batch_ref[0, 0, :]
    onehot = (bvec[:, None]
              == lax.broadcasted_iota(jnp.int32, (1, G), 1)).astype(
                  jnp.bfloat16)
    sums_ref[...] += lax.dot_general(
        onehot, h_ref[...], (((0,), (0,)), ((), ())),
        preferred_element_type=jnp.float32)
    cnt_ref[...] += jnp.broadcast_to(
        jnp.sum(onehot.astype(jnp.float32), axis=0)[:, None], (G, 128))

    @pl.when(i == (NPAD // MBLK) - 1)
    def _fin():
        cnt = jnp.clip(cnt_ref[:, 0:1], 1.0, None)
        out_ref[...] = sums_ref[...] / cnt


def _pool(batch3, h2):
    _, _, out = pl.pallas_call(
        _pool_body,
        grid=(NPAD // MBLK,),
        in_specs=[
            pl.BlockSpec((1, 1, MBLK), lambda i: (i, 0, 0)),
            pl.BlockSpec((MBLK, HC), lambda i: (i, 0)),
        ],
        out_specs=[
            pl.BlockSpec((G, HC), lambda i: (0, 0)),
            pl.BlockSpec((G, 128), lambda i: (0, 0)),
            pl.BlockSpec((G, HC), lambda i: (0, 0)),
        ],
        out_shape=[
            jax.ShapeDtypeStruct((G, HC), jnp.float32),
            jax.ShapeDtypeStruct((G, 128), jnp.float32),
            jax.ShapeDtypeStruct((G, HC), jnp.float32),
        ],
    )(batch3, h2)
    return out


def kernel(x, edge_index, edge_attr, batch, Wl1, bl1, Wr1, br1, We1, att1, b1,
           Wl2, bl2, Wr2, br2, We2, att2, b2):
    src = edge_index[0]
    dst = edge_index[1]

    perm = jnp.argsort(dst)
    srcs = src[perm]
    dsts = dst[perm]
    eas = edge_attr[perm, 0]
    blk = dsts // B
    estart = jnp.searchsorted(dsts, (jnp.arange(NB + 1) * B).astype(jnp.int32)
                              ).astype(jnp.int32)
    cnt = estart[1:] - estart[:-1]
    ngrp = (cnt + GSZ - 1) // GSZ
    pstart = jnp.concatenate([jnp.zeros((1,), jnp.int32),
                              jnp.cumsum(ngrp * GSZ).astype(jnp.int32)])
    pos = pstart[blk] + (jnp.arange(E, dtype=jnp.int32) - estart[blk])
    psrc = jnp.zeros((EP,), jnp.int32).at[pos].set(srcs)
    pdl = jnp.full((EP,), B, jnp.int32).at[pos].set(
        (dsts - blk * B).astype(jnp.int32))
    pea = jnp.zeros((EP,), jnp.float32).at[pos].set(eas)
    eau = lax.bitcast_convert_type(pea.astype(jnp.bfloat16),
                                   jnp.uint16).astype(jnp.uint32)
    pea_bits = (eau | (eau << 16)).astype(jnp.int32)
    pmeta = jnp.concatenate([
        psrc.reshape(EPG, 1, GSZ),
        pdl.reshape(EPG, 1, GSZ),
        pea_bits.reshape(EPG, 1, GSZ),
    ], axis=1)
    est2 = jnp.zeros((NBP, 16), jnp.int32)
    est2 = (est2.at[:NB, 0].set(pstart[:NB])
                .at[:NB, 1].set(ngrp)
                .at[:NB, 2].set(1))

    wat1 = jnp.stack([We1[0], att1.reshape(-1)]).astype(jnp.bfloat16)
    wat2 = jnp.stack([We2[0], att2.reshape(-1)]).astype(jnp.bfloat16)

    xp = jnp.zeros((NPAD, D), jnp.bfloat16).at[:N].set(x.astype(jnp.bfloat16))
    xl1, xr1 = _proj(xp, Wl1.astype(jnp.bfloat16), Wr1.astype(jnp.bfloat16),
                     bl1, br1)
    h = _sc_edge(xl1, xr1, pmeta, est2, wat1, b1[IPERM])

    xl2, xr2 = _proj(h, Wl2.astype(jnp.bfloat16), Wr2.astype(jnp.bfloat16),
                     bl2, br2)
    h2 = _sc_edge(xl2, xr2, pmeta, est2, wat2, b2[IPERM])

    batch3 = jnp.full((NPAD,), G, jnp.int32).at[:N].set(batch).reshape(
        NPAD // MBLK, 1, MBLK)
    return _pool(batch3, h2)


# speedup vs baseline: 2.2896x; 1.0206x over previous
import numpy as np

import jax
import jax.numpy as jnp
from jax import lax
from jax.experimental import pallas as pl
from jax.experimental.pallas import tpu as pltpu
from jax.experimental.pallas import tpu_sc as plsc

N = 10000
E = 32000
D = 2560
H = 8
C = 320
HC = H * C
G = 8

NPAD = 10240
MBLK = 256

B = 8
NB = N // B
NBP = NPAD // B
GSZ = 16
EP = E + GSZ * NB
EPG = EP // GSZ
NCHUNK = HC // 16
CP2 = C // 32

_p = np.arange(HC)
IPERM = ((_p // 32) * 32 + (_p % 2) * 16 + (_p % 32) // 2).astype(np.int32)
INV_IPERM = np.argsort(IPERM).astype(np.int32)

_ILV = plsc.PackFormat.INTERLEAVED


def _mm_body(x_ref, wl_ref, wr_ref, bl_ref, br_ref, xl_ref, xr_ref):
    x = x_ref[...]
    xl_ref[...] = (
        jax.lax.dot(x, wl_ref[...], preferred_element_type=jnp.float32)
        + bl_ref[...]
    ).astype(jnp.bfloat16)
    xr_ref[...] = (
        jax.lax.dot(x, wr_ref[...], preferred_element_type=jnp.float32)
        + br_ref[...]
    ).astype(jnp.bfloat16)


def _proj(x_bf16, wl, wr, bl, br):
    k = x_bf16.shape[1]
    return pl.pallas_call(
        _mm_body,
        grid=(NPAD // MBLK,),
        in_specs=[
            pl.BlockSpec((MBLK, k), lambda i: (i, 0)),
            pl.BlockSpec((k, HC), lambda i: (0, 0)),
            pl.BlockSpec((k, HC), lambda i: (0, 0)),
            pl.BlockSpec((1, HC), lambda i: (0, 0)),
            pl.BlockSpec((1, HC), lambda i: (0, 0)),
        ],
        out_specs=[
            pl.BlockSpec((MBLK, HC), lambda i: (i, 0)),
            pl.BlockSpec((MBLK, HC), lambda i: (i, 0)),
        ],
        out_shape=[
            jax.ShapeDtypeStruct((NPAD, HC), jnp.bfloat16),
            jax.ShapeDtypeStruct((NPAD, HC), jnp.bfloat16),
        ],
    )(x_bf16, wl, wr, bl[None, :], br[None, :])


def _sc_edge_body(xl_h, xr_h, pmeta_h, est2_h, wat_h, bias_h, out_h,
                  est_s, xr_s, acc_s, asum_s, rec_s, xlg_s, meta_s, stage_s,
                  wat_s, bias_s, gsem):
    ncores = 2
    nw = 32
    nbt = NBP // nw

    wid = lax.axis_index("s") * ncores + lax.axis_index("c")

    pltpu.sync_copy(est2_h, est_s)
    pltpu.sync_copy(wat_h, wat_s)
    pltpu.sync_copy(bias_h, bias_s)

    zero16 = jnp.zeros((16,), jnp.float32)
    zero32b = jnp.zeros((32,), jnp.bfloat16)

    def _zxr(j, _):
        xr_s[B, pl.ds(j * 32, 32)] = zero32b
        return 0
    lax.fori_loop(0, HC // 32, _zxr, 0)

    lane = lax.broadcasted_iota(jnp.int32, (GSZ,), 0)

    def block_body(bi, _):
        b = bi * nw + wid
        ev = est_s[b]
        p0 = ev[0]
        ngb = ev[1]
        realf = ev[2].astype(jnp.float32)
        gi0 = lax.shift_right_logical(p0, 4)
        n0 = b * B

        def _zacc(r, _):
            def _zc(k, __):
                acc_s[r, pl.ds(k * 16, 16)] = zero16
                return 0
            lax.fori_loop(0, NCHUNK, _zc, 0)
            asum_s[r] = zero16
            return 0
        lax.fori_loop(0, B + 1, _zacc, 0)

        pltpu.sync_copy(xr_h.at[pl.ds(n0, B)], xr_s.at[pl.ds(0, B)])

        @pl.when(ngb > 0)
        def _prol():
            pltpu.sync_copy(pmeta_h.at[gi0], meta_s.at[0])
            pltpu.async_copy(xl_h.at[meta_s.at[0, 0]], xlg_s.at[0], gsem)

        def group_body(g, _):
            par = lax.bitwise_and(g, 1)
            pltpu.make_async_copy(xl_h.at[meta_s.at[par, 0]],
                                  xlg_s.at[par], gsem).wait()

            @pl.when(g + 1 < ngb)
            def _prefetch():
                npar = lax.bitwise_and(g + 1, 1)
                pltpu.sync_copy(pmeta_h.at[gi0 + g + 1], meta_s.at[npar])
                pltpu.async_copy(xl_h.at[meta_s.at[npar, 0]],
                                 xlg_s.at[npar], gsem)

            def edge_body(i, _):
                iv = jnp.full((GSZ,), i, jnp.int32)
                dl = plsc.load_gather(meta_s.at[par, 1], [iv])[0]
                eab = plsc.bitcast(plsc.load_gather(meta_s.at[par, 2], [iv]),
                                   jnp.bfloat16)

                def _logit_term(o, accv):
                    z = (xlg_s[par, i, pl.ds(o, 32)]
                         + xr_s[dl, pl.ds(o, 32)]
                         + eab * wat_s[0, pl.ds(o, 32)])
                    z = jnp.maximum(z, 0.2 * z)
                    za, zb = plsc.unpack(z, format=_ILV)
                    aa, ab = plsc.unpack(wat_s[1, pl.ds(o, 32)], format=_ILV)
                    return accv + za * aa + zb * ab

                def _msg_term(o, ah):
                    la, lb = plsc.unpack(xlg_s[par, i, pl.ds(o, 32)],
                                         format=_ILV)
                    plsc.addupdate(acc_s.at[dl, pl.ds(o, 16)], ah * la)
                    plsc.addupdate(acc_s.at[dl, pl.ds(o + 16, 16)], ah * lb)

                def _exp16(accv):
                    return jnp.exp(jnp.full((GSZ,), jnp.sum(accv),
                                            jnp.float32))[0]

                acc0 = lax.fori_loop(
                    0, CP2, lambda j, a: _logit_term(j * 32, a), zero16)
                ah0 = _exp16(acc0)

                def chain(h, ahp):
                    hoff = h * C

                    def _f(j, accv):
                        _msg_term(hoff - C + j * 64, ahp)
                        accv = _logit_term(hoff + j * 64, accv)
                        _msg_term(hoff - C + j * 64 + 32, ahp)
                        return _logit_term(hoff + j * 64 + 32, accv)
                    accv = lax.fori_loop(0, CP2 // 2, _f, zero16)
                    plsc.addupdate(asum_s.at[dl],
                                   jnp.where(lane == h - 1, ahp, 0.0))
                    return _exp16(accv)
                ah7 = lax.fori_loop(1, H, chain, ah0)

                def _m7(j, __):
                    _msg_term((H - 1) * C + j * 32, ah7)
                    return 0
                lax.fori_loop(0, CP2, _m7, 0)
                plsc.addupdate(asum_s.at[dl],
                               jnp.where(lane == H - 1, ah7, 0.0))
                return 0
            lax.fori_loop(0, GSZ, edge_body, 0)
            return 0
        lax.fori_loop(0, ngb, group_body, 0)

        def _rec(r, _):
            rec_s[r] = 1.0 / (asum_s[r] + 1e-16)
            return 0
        lax.fori_loop(0, B, _rec, 0)

        def _norm_r(r, _):
            rv16 = jnp.full((GSZ,), r, jnp.int32)

            def _norm_h(h, __):
                rv = plsc.load_gather(rec_s, [rv16, jnp.full((GSZ,), h,
                                                             jnp.int32)])[0]
                hoff = h * C

                def _norm_j(j, ___):
                    o = hoff + j * 32
                    va = (acc_s[r, pl.ds(o, 16)] * rv
                          + realf * bias_s[pl.ds(o, 16)])
                    vb = (acc_s[r, pl.ds(o + 16, 16)] * rv
                          + realf * bias_s[pl.ds(o + 16, 16)])
                    stage_s[r, pl.ds(o, 32)] = plsc.pack(va, vb, format=_ILV)
                    return 0
                lax.fori_loop(0, CP2, _norm_j, 0)
                return 0
            lax.fori_loop(0, H, _norm_h, 0)
            return 0
        lax.fori_loop(0, B, _norm_r, 0)

        pltpu.sync_copy(stage_s, out_h.at[pl.ds(n0, B)])
        return 0
    lax.fori_loop(0, nbt, block_body, 0)


def _sc_edge(xl, xr, pmeta, est2, wat, bias):
    mesh = plsc.VectorSubcoreMesh(core_axis_name="c", subcore_axis_name="s")
    return pl.kernel(
        _sc_edge_body,
        out_type=jax.ShapeDtypeStruct((NPAD, HC), jnp.bfloat16),
        mesh=mesh,
        compiler_params=pltpu.CompilerParams(needs_layout_passes=False,
                                             use_tc_tiling_on_sc=False),
        scratch_types=[
            pltpu.VMEM((NBP, 16), jnp.int32),
            pltpu.VMEM((B + 1, HC), jnp.bfloat16),
            pltpu.VMEM((B + 1, HC), jnp.float32),
            pltpu.VMEM((B + 1, 16), jnp.float32),
            pltpu.VMEM((B, 16), jnp.float32),
            pltpu.VMEM((2, GSZ, HC), jnp.bfloat16),
            pltpu.VMEM((2, 3, 16), jnp.int32),
            pltpu.VMEM((B, HC), jnp.bfloat16),
            pltpu.VMEM((2, HC), jnp.bfloat16),
            pltpu.VMEM((HC,), jnp.float32),
            pltpu.SemaphoreType.DMA,
        ],
    )(xl, xr, pmeta, est2, wat, bias)


def _pool_body(batch_ref, h_ref, sums_ref, cnt_ref, out_ref):
    i = pl.program_id(0)

    @pl.when(i == 0)
    def _init():
        sums_ref[...] = jnp.zeros_like(sums_ref)
        cnt_ref[...] = jnp.zeros_like(cnt_ref)

    bvec = batch_ref[0, 0, :]
    onehot = (bvec[:, None]
              == lax.broadcasted_iota(jnp.int32, (1, G), 1)).astype(
                  jnp.bfloat16)
    sums_ref[...] += lax.dot_general(
        onehot, h_ref[...], (((0,), (0,)), ((), ())),
        preferred_element_type=jnp.float32)
    cnt_ref[...] += jnp.broadcast_to(
        jnp.sum(onehot.astype(jnp.float32), axis=0)[:, None], (G, 128))

    @pl.when(i == (NPAD // MBLK) - 1)
    def _fin():
        cnt = jnp.clip(cnt_ref[:, 0:1], 1.0, None)
        out_ref[...] = sums_ref[...] / cnt


def _pool(batch3, h2):
    _, _, out = pl.pallas_call(
        _pool_body,
        grid=(NPAD // MBLK,),
        in_specs=[
            pl.BlockSpec((1, 1, MBLK), lambda i: (i, 0, 0)),
            pl.BlockSpec((MBLK, HC), lambda i: (i, 0)),
        ],
        out_specs=[
            pl.BlockSpec((G, HC), lambda i: (0, 0)),
            pl.BlockSpec((G, 128), lambda i: (0, 0)),
            pl.BlockSpec((G, HC), lambda i: (0, 0)),
        ],
        out_shape=[
            jax.ShapeDtypeStruct((G, HC), jnp.float32),
            jax.ShapeDtypeStruct((G, 128), jnp.float32),
            jax.ShapeDtypeStruct((G, HC), jnp.float32),
        ],
    )(batch3, h2)
    return out


def kernel(x, edge_index, edge_attr, batch, Wl1, bl1, Wr1, br1, We1, att1, b1,
           Wl2, bl2, Wr2, br2, We2, att2, b2):
    src = edge_index[0]
    dst = edge_index[1]

    perm = jnp.argsort(dst)
    srcs = src[perm]
    dsts = dst[perm]
    eas = edge_attr[perm, 0]
    blk = dsts // B
    estart = jnp.searchsorted(dsts, (jnp.arange(NB + 1) * B).astype(jnp.int32)
                              ).astype(jnp.int32)
    cnt = estart[1:] - estart[:-1]
    ngrp = (cnt + GSZ - 1) // GSZ
    pstart = jnp.concatenate([jnp.zeros((1,), jnp.int32),
                              jnp.cumsum(ngrp * GSZ).astype(jnp.int32)])
    pos = pstart[blk] + (jnp.arange(E, dtype=jnp.int32) - estart[blk])
    psrc = jnp.zeros((EP,), jnp.int32).at[pos].set(srcs)
    pdl = jnp.full((EP,), B, jnp.int32).at[pos].set(
        (dsts - blk * B).astype(jnp.int32))
    pea = jnp.zeros((EP,), jnp.float32).at[pos].set(eas)
    eau = lax.bitcast_convert_type(pea.astype(jnp.bfloat16),
                                   jnp.uint16).astype(jnp.uint32)
    pea_bits = (eau | (eau << 16)).astype(jnp.int32)
    pmeta = jnp.concatenate([
        psrc.reshape(EPG, 1, GSZ),
        pdl.reshape(EPG, 1, GSZ),
        pea_bits.reshape(EPG, 1, GSZ),
    ], axis=1)
    est2 = jnp.zeros((NBP, 16), jnp.int32)
    est2 = (est2.at[:NB, 0].set(pstart[:NB])
                .at[:NB, 1].set(ngrp)
                .at[:NB, 2].set(1))

    wat1 = jnp.stack([We1[0], att1.reshape(-1)]).astype(jnp.bfloat16)
    wat2 = jnp.stack([We2[0], att2.reshape(-1)]).astype(jnp.bfloat16)

    xp = jnp.zeros((NPAD, D), jnp.bfloat16).at[:N].set(x.astype(jnp.bfloat16))
    xl1, xr1 = _proj(xp, Wl1.astype(jnp.bfloat16), Wr1.astype(jnp.bfloat16),
                     bl1, br1)
    h = _sc_edge(xl1, xr1, pmeta, est2, wat1, b1[IPERM])

    xl2, xr2 = _proj(h, Wl2.astype(jnp.bfloat16), Wr2.astype(jnp.bfloat16),
                     bl2, br2)
    h2 = _sc_edge(xl2, xr2, pmeta, est2, wat2, b2[IPERM])

    batch3 = jnp.full((NPAD,), G, jnp.int32).at[:N].set(batch).reshape(
        NPAD // MBLK, 1, MBLK)
    return _pool(batch3, h2)
